# baseline (speedup 1.0000x reference)
import jax
import jax.numpy as jnp
from jax import lax
from jax.experimental import pallas as pl
from jax.experimental.pallas import tpu as pltpu
from jax.experimental.pallas import tpu_sc as plsc

H = 768
I = 1536
E = 63
EP = 64
TOPK = 2
N = 2048
A = N * TOPK
M = 512
T = A // M
G = T + E - 1
C = 128
NCHUNK = A // C

_NT = (((1,), (1,)), ((), ()))


def _dot_nt(a, b):
  return lax.dot_general(a, b, _NT, preferred_element_type=jnp.float32)


def _plan_body(x_ref, rw_ref, b_ref, meta_ref, rank_ref, w1_ref, w2_ref,
               e12_ref):
  x = x_ref[...]
  logits = _dot_nt(x, rw_ref[...])
  logits = logits + b_ref[...]
  probs = jax.nn.sigmoid(logits)
  eidx = lax.broadcasted_iota(jnp.int32, probs.shape, 1)
  big = jnp.int32(2 ** 30)
  m1 = jnp.max(probs, axis=1, keepdims=True)
  i1 = jnp.min(jnp.where(probs == m1, eidx, big), axis=1, keepdims=True)
  probs2 = jnp.where(eidx == i1, -jnp.inf, probs)
  m2 = jnp.max(probs2, axis=1, keepdims=True)
  i2 = jnp.min(jnp.where(probs2 == m2, eidx, big), axis=1, keepdims=True)
  denom = m1 + m2
  w1_ref[...] = m1 / denom
  w2_ref[...] = m2 / denom
  e12_ref[0:N, :] = i1
  e12_ref[N:A, :] = i2

  oh1 = (i1 == eidx).astype(jnp.float32)
  oh2 = (i2 == eidx).astype(jnp.float32)
  counts = (jnp.sum(oh1, axis=0, keepdims=True)
            + jnp.sum(oh2, axis=0, keepdims=True))

  er = lax.broadcasted_iota(jnp.int32, (EP, EP), 0)
  ec = lax.broadcasted_iota(jnp.int32, (EP, EP), 1)
  tri_e = (er < ec).astype(jnp.float32)
  starts = _dot_nt(counts, tri_e.T)
  ends = starts + counts

  rr = lax.broadcasted_iota(jnp.int32, (C, C), 0)
  rc = lax.broadcasted_iota(jnp.int32, (C, C), 1)
  tri_c = (rc < rr).astype(jnp.float32)
  ch_eidx = lax.broadcasted_iota(jnp.int32, (C, EP), 1)

  def chunk(c, running):
    e_ch = e12_ref[pl.ds(c * C, C), :]
    oh = (e_ch == ch_eidx).astype(jnp.float32)
    prefix = lax.dot_general(tri_c, oh, (((1,), (0,)), ((), ())),
                             preferred_element_type=jnp.float32)
    base = starts + running
    r = jnp.sum(oh * (base + prefix), axis=1, keepdims=True)
    rank_ref[pl.ds(c * C, C), :] = r.astype(jnp.int32)
    return running + jnp.sum(oh, axis=0, keepdims=True)

  lax.fori_loop(0, NCHUNK, chunk, jnp.zeros((1, EP), jnp.float32))

  startsc = starts.reshape(EP, 1)
  countsc = counts.reshape(EP, 1)
  endsc = ends.reshape(EP, 1)
  ftile = jnp.floor(startsc / M)
  ltile = jnp.floor((endsc - 1.0) / M)
  ntiles = jnp.where(countsc > 0.0, ltile - ftile + 1.0, 0.0)
  woff = _dot_nt(ntiles.reshape(1, EP), tri_e.T).reshape(EP, 1)
  total = jnp.sum(ntiles)

  gl = lax.broadcasted_iota(jnp.int32, (1, 128), 1).astype(jnp.float32)
  le_mask = (woff <= gl).astype(jnp.float32)
  eg = jnp.sum(le_mask, axis=0, keepdims=True) - 1.0
  ee = lax.broadcasted_iota(jnp.int32, (EP, 128), 0).astype(jnp.float32)
  oh_g = (ee == eg).astype(jnp.float32)
  tg = jnp.sum(oh_g * ftile, axis=0, keepdims=True) + gl - \
      jnp.sum(oh_g * woff, axis=0, keepdims=True)
  lo = jnp.sum(oh_g * startsc, axis=0, keepdims=True)
  hi = jnp.sum(oh_g * endsc, axis=0, keepdims=True)
  valid = (gl < total).astype(jnp.float32)
  lastsel = (gl == total - 1.0).astype(jnp.float32)
  last_t = jnp.sum(lastsel * tg, axis=1, keepdims=True)
  last_e = jnp.sum(lastsel * eg, axis=1, keepdims=True)
  tg = valid * tg + (1.0 - valid) * last_t
  eg = valid * eg + (1.0 - valid) * last_e
  lo = valid * lo
  hi = valid * hi
  tg_prev = jnp.concatenate([jnp.full((1, 1), -1.0, jnp.float32),
                             tg[:, :127]], axis=1)
  fv = valid * (tg != tg_prev).astype(jnp.float32)
  meta = jnp.concatenate(
      [tg, eg, lo, hi, fv, jnp.zeros((3, 128), jnp.float32)], axis=0)
  meta_ref[...] = meta.astype(jnp.int32)


def _plan(flat, router_w, routing_bias):
  rw = jnp.zeros((EP, H), jnp.float32).at[:E].set(router_w)
  bias = jnp.full((1, EP), -jnp.inf, jnp.float32).at[0, :E].set(routing_bias)
  return pl.pallas_call(
      _plan_body,
      out_shape=(
          jax.ShapeDtypeStruct((8, 128), jnp.int32),
          jax.ShapeDtypeStruct((A, 1), jnp.int32),
          jax.ShapeDtypeStruct((N, 1), jnp.float32),
          jax.ShapeDtypeStruct((N, 1), jnp.float32),
      ),
      scratch_shapes=[pltpu.VMEM((A, 1), jnp.int32)],
  )(flat, rw, bias)


def _dispatch_body(flat_hbm, rank_hbm, xs_hbm, idx_v, rows_v, sem):
  info = plsc.get_sparse_core_info()
  nc = info.num_cores
  wid = lax.axis_index("s") * nc + lax.axis_index("c")
  rpw = A // (nc * info.num_subcores)
  base = wid * rpw
  tok_base = lax.rem(base, N)
  pltpu.sync_copy(rank_hbm.at[pl.ds(base, rpw)], idx_v)
  pltpu.sync_copy(flat_hbm.at[pl.ds(tok_base, rpw)], rows_v)
  pltpu.async_copy(rows_v, xs_hbm.at[idx_v], sem).wait()


def _dispatch(flat, rank):
  info = plsc.get_sparse_core_info()
  rpw = A // (info.num_cores * info.num_subcores)
  mesh = plsc.VectorSubcoreMesh(core_axis_name="c", subcore_axis_name="s")
  return pl.kernel(
      _dispatch_body,
      out_type=jax.ShapeDtypeStruct((A, H), jnp.float32),
      mesh=mesh,
      scratch_types=[
          pltpu.VMEM((rpw,), jnp.int32),
          pltpu.VMEM((rpw, H), jnp.float32),
          pltpu.SemaphoreType.DMA,
      ],
  )(flat, rank)


def _grouped_body(meta_ref, xs_ref, gw_ref, uw_ref, dw_ref, ys_ref):
  g = pl.program_id(0)
  t = meta_ref[0, g]
  lo = meta_ref[2, g]
  hi = meta_ref[3, g]
  fv = meta_ref[4, g]

  @pl.when(fv == 1)
  def _():
    ys_ref[...] = jnp.zeros_like(ys_ref)

  @pl.when(hi > lo)
  def _():
    rows = t * M + lax.broadcasted_iota(jnp.int32, (M, 1), 0)
    mask = (rows >= lo) & (rows < hi)
    x = xs_ref[...]
    hpre = _dot_nt(x, gw_ref[0])
    h = jax.nn.silu(hpre) * _dot_nt(x, uw_ref[0])
    contrib = _dot_nt(h, dw_ref[0])
    ys_ref[...] += jnp.where(mask, contrib, 0.0)


def _grouped(meta, xs, routed_gate, routed_up, routed_down):
  grid_spec = pltpu.PrefetchScalarGridSpec(
      num_scalar_prefetch=1,
      grid=(G,),
      in_specs=[
          pl.BlockSpec((M, H), lambda g, meta: (meta[0, g], 0)),
          pl.BlockSpec((1, I, H), lambda g, meta: (meta[1, g], 0, 0)),
          pl.BlockSpec((1, I, H), lambda g, meta: (meta[1, g], 0, 0)),
          pl.BlockSpec((1, H, I), lambda g, meta: (meta[1, g], 0, 0)),
      ],
      out_specs=pl.BlockSpec((M, H), lambda g, meta: (meta[0, g], 0)),
  )
  return pl.pallas_call(
      _grouped_body,
      grid_spec=grid_spec,
      out_shape=jax.ShapeDtypeStruct((A, H), jnp.float32),
      compiler_params=pltpu.CompilerParams(
          dimension_semantics=("arbitrary",),
          vmem_limit_bytes=128 * 1024 * 1024,
      ),
  )(meta, xs, routed_gate, routed_up, routed_down)


def _combine_body(ys_hbm, inv1_hbm, inv2_hbm, y1_hbm, y2_hbm,
                  i1_v, i2_v, r1_v, r2_v, sem1, sem2):
  info = plsc.get_sparse_core_info()
  nc = info.num_cores
  wid = lax.axis_index("s") * nc + lax.axis_index("c")
  rpw = N // (nc * info.num_subcores)
  base = wid * rpw
  pltpu.sync_copy(inv1_hbm.at[pl.ds(base, rpw)], i1_v)
  pltpu.sync_copy(inv2_hbm.at[pl.ds(base, rpw)], i2_v)
  cp1 = pltpu.async_copy(ys_hbm.at[i1_v], r1_v, sem1)
  cp2 = pltpu.async_copy(ys_hbm.at[i2_v], r2_v, sem2)
  cp1.wait()
  cp2.wait()
  pltpu.sync_copy(r1_v, y1_hbm.at[pl.ds(base, rpw)])
  pltpu.sync_copy(r2_v, y2_hbm.at[pl.ds(base, rpw)])


def _combine(ys, inv1, inv2):
  info = plsc.get_sparse_core_info()
  rpw = N // (info.num_cores * info.num_subcores)
  mesh = plsc.VectorSubcoreMesh(core_axis_name="c", subcore_axis_name="s")
  return pl.kernel(
      _combine_body,
      out_type=(
          jax.ShapeDtypeStruct((N, H), jnp.float32),
          jax.ShapeDtypeStruct((N, H), jnp.float32),
      ),
      mesh=mesh,
      scratch_types=[
          pltpu.VMEM((rpw,), jnp.int32),
          pltpu.VMEM((rpw,), jnp.int32),
          pltpu.VMEM((rpw, H), jnp.float32),
          pltpu.VMEM((rpw, H), jnp.float32),
          pltpu.SemaphoreType.DMA,
          pltpu.SemaphoreType.DMA,
      ],
  )(ys, inv1, inv2)


MS = 256


def _shared_body(x_ref, sg_ref, su_ref, sd_ref, y1_ref, y2_ref,
                 w1_ref, w2_ref, o_ref):
  t = pl.program_id(0)
  x = x_ref[...]
  h = jax.nn.silu(_dot_nt(x, sg_ref[...])) * _dot_nt(x, su_ref[...])
  out = _dot_nt(h, sd_ref[...])
  w1 = w1_ref[pl.ds(t * MS, MS), :]
  w2 = w2_ref[pl.ds(t * MS, MS), :]
  o_ref[...] = out + w1 * y1_ref[...] + w2 * y2_ref[...]


def _shared_combine(flat, sg, su, sd, ys1, ys2, w1, w2):
  nt = N // MS
  return pl.pallas_call(
      _shared_body,
      grid=(nt,),
      in_specs=[
          pl.BlockSpec((MS, H), lambda t: (t, 0)),
          pl.BlockSpec((I, H), lambda t: (0, 0)),
          pl.BlockSpec((I, H), lambda t: (0, 0)),
          pl.BlockSpec((H, I), lambda t: (0, 0)),
          pl.BlockSpec((MS, H), lambda t: (t, 0)),
          pl.BlockSpec((MS, H), lambda t: (t, 0)),
          pl.BlockSpec((N, 1), lambda t: (0, 0)),
          pl.BlockSpec((N, 1), lambda t: (0, 0)),
      ],
      out_specs=pl.BlockSpec((MS, H), lambda t: (t, 0)),
      out_shape=jax.ShapeDtypeStruct((N, H), jnp.float32),
      compiler_params=pltpu.CompilerParams(
          dimension_semantics=("arbitrary",),
          vmem_limit_bytes=128 * 1024 * 1024,
      ),
  )(flat, sg, su, sd, ys1, ys2, w1, w2)


def kernel(x, shared_gate, shared_up, shared_down, routed_gate, routed_up,
           routed_down, router_w, routing_bias):
  flat = x.reshape(N, H)
  meta, rank, w1, w2 = _plan(flat, router_w, routing_bias)
  rank_flat = rank.reshape(A)
  xs = _dispatch(flat, rank_flat)
  ys = _grouped(meta, xs, routed_gate, routed_up, routed_down)
  ys1, ys2 = _combine(ys, rank_flat[:N], rank_flat[N:])
  out = _shared_combine(flat, shared_gate, shared_up, shared_down,
                        ys1, ys2, w1, w2)
  return out.reshape(x.shape)


# speedup vs baseline: 1.0295x; 1.0295x over previous
import jax
import jax.numpy as jnp
from jax import lax
from jax.experimental import pallas as pl
from jax.experimental.pallas import tpu as pltpu
from jax.experimental.pallas import tpu_sc as plsc

H = 768
I = 1536
E = 63
EP = 64
TOPK = 2
N = 2048
A = N * TOPK
M = 256
T = A // M
G = T + E - 1
C = 128
NCHUNK = A // C

_NT = (((1,), (1,)), ((), ()))


def _dot_nt(a, b):
  return lax.dot_general(a, b, _NT, preferred_element_type=jnp.float32)


def _plan_body(x_ref, rw_ref, b_ref, meta_ref, rank_ref, w1_ref, w2_ref,
               e12_ref):
  x = x_ref[...]
  logits = _dot_nt(x, rw_ref[...])
  logits = logits + b_ref[...]
  probs = jax.nn.sigmoid(logits)
  eidx = lax.broadcasted_iota(jnp.int32, probs.shape, 1)
  big = jnp.int32(2 ** 30)
  m1 = jnp.max(probs, axis=1, keepdims=True)
  i1 = jnp.min(jnp.where(probs == m1, eidx, big), axis=1, keepdims=True)
  probs2 = jnp.where(eidx == i1, -jnp.inf, probs)
  m2 = jnp.max(probs2, axis=1, keepdims=True)
  i2 = jnp.min(jnp.where(probs2 == m2, eidx, big), axis=1, keepdims=True)
  denom = m1 + m2
  w1_ref[...] = m1 / denom
  w2_ref[...] = m2 / denom
  e12_ref[0:N, :] = i1
  e12_ref[N:A, :] = i2

  oh1 = (i1 == eidx).astype(jnp.float32)
  oh2 = (i2 == eidx).astype(jnp.float32)
  counts = (jnp.sum(oh1, axis=0, keepdims=True)
            + jnp.sum(oh2, axis=0, keepdims=True))

  er = lax.broadcasted_iota(jnp.int32, (EP, EP), 0)
  ec = lax.broadcasted_iota(jnp.int32, (EP, EP), 1)
  tri_e = (er < ec).astype(jnp.float32)
  starts = _dot_nt(counts, tri_e.T)
  ends = starts + counts

  rr = lax.broadcasted_iota(jnp.int32, (C, C), 0)
  rc = lax.broadcasted_iota(jnp.int32, (C, C), 1)
  tri_c = (rc < rr).astype(jnp.float32)
  ch_eidx = lax.broadcasted_iota(jnp.int32, (C, EP), 1)

  def chunk(c, running):
    e_ch = e12_ref[pl.ds(c * C, C), :]
    oh = (e_ch == ch_eidx).astype(jnp.float32)
    prefix = lax.dot_general(tri_c, oh, (((1,), (0,)), ((), ())),
                             preferred_element_type=jnp.float32)
    base = starts + running
    r = jnp.sum(oh * (base + prefix), axis=1, keepdims=True)
    rank_ref[pl.ds(c * C, C), :] = r.astype(jnp.int32)
    return running + jnp.sum(oh, axis=0, keepdims=True)

  lax.fori_loop(0, NCHUNK, chunk, jnp.zeros((1, EP), jnp.float32))

  startsc = starts.reshape(EP, 1)
  countsc = counts.reshape(EP, 1)
  endsc = ends.reshape(EP, 1)
  ftile = jnp.floor(startsc / M)
  ltile = jnp.floor((endsc - 1.0) / M)
  ntiles = jnp.where(countsc > 0.0, ltile - ftile + 1.0, 0.0)
  woff = _dot_nt(ntiles.reshape(1, EP), tri_e.T).reshape(EP, 1)
  total = jnp.sum(ntiles)

  gl = lax.broadcasted_iota(jnp.int32, (1, 128), 1).astype(jnp.float32)
  le_mask = (woff <= gl).astype(jnp.float32)
  eg = jnp.sum(le_mask, axis=0, keepdims=True) - 1.0
  ee = lax.broadcasted_iota(jnp.int32, (EP, 128), 0).astype(jnp.float32)
  oh_g = (ee == eg).astype(jnp.float32)
  tg = jnp.sum(oh_g * ftile, axis=0, keepdims=True) + gl - \
      jnp.sum(oh_g * woff, axis=0, keepdims=True)
  lo = jnp.sum(oh_g * startsc, axis=0, keepdims=True)
  hi = jnp.sum(oh_g * endsc, axis=0, keepdims=True)
  valid = (gl < total).astype(jnp.float32)
  lastsel = (gl == total - 1.0).astype(jnp.float32)
  last_t = jnp.sum(lastsel * tg, axis=1, keepdims=True)
  last_e = jnp.sum(lastsel * eg, axis=1, keepdims=True)
  tg = valid * tg + (1.0 - valid) * last_t
  eg = valid * eg + (1.0 - valid) * last_e
  lo = valid * lo
  hi = valid * hi
  tg_prev = jnp.concatenate([jnp.full((1, 1), -1.0, jnp.float32),
                             tg[:, :127]], axis=1)
  fv = valid * (tg != tg_prev).astype(jnp.float32)
  meta = jnp.concatenate(
      [tg, eg, lo, hi, fv, jnp.zeros((3, 128), jnp.float32)], axis=0)
  meta_ref[...] = meta.astype(jnp.int32)


def _plan(flat, router_w, routing_bias):
  rw = jnp.zeros((EP, H), jnp.float32).at[:E].set(router_w)
  bias = jnp.full((1, EP), -jnp.inf, jnp.float32).at[0, :E].set(routing_bias)
  return pl.pallas_call(
      _plan_body,
      out_shape=(
          jax.ShapeDtypeStruct((8, 128), jnp.int32),
          jax.ShapeDtypeStruct((A, 1), jnp.int32),
          jax.ShapeDtypeStruct((N, 1), jnp.float32),
          jax.ShapeDtypeStruct((N, 1), jnp.float32),
      ),
      scratch_shapes=[pltpu.VMEM((A, 1), jnp.int32)],
  )(flat, rw, bias)


def _dispatch_body(flat_hbm, rank_hbm, xs_hbm, idx_v, rows_v, sem):
  info = plsc.get_sparse_core_info()
  nc = info.num_cores
  wid = lax.axis_index("s") * nc + lax.axis_index("c")
  rpw = A // (nc * info.num_subcores)
  base = wid * rpw
  tok_base = lax.rem(base, N)
  pltpu.sync_copy(rank_hbm.at[pl.ds(base, rpw)], idx_v)
  pltpu.sync_copy(flat_hbm.at[pl.ds(tok_base, rpw)], rows_v)
  pltpu.async_copy(rows_v, xs_hbm.at[idx_v], sem).wait()


def _dispatch(flat, rank):
  info = plsc.get_sparse_core_info()
  rpw = A // (info.num_cores * info.num_subcores)
  mesh = plsc.VectorSubcoreMesh(core_axis_name="c", subcore_axis_name="s")
  return pl.kernel(
      _dispatch_body,
      out_type=jax.ShapeDtypeStruct((A, H), jnp.float32),
      mesh=mesh,
      scratch_types=[
          pltpu.VMEM((rpw,), jnp.int32),
          pltpu.VMEM((rpw, H), jnp.float32),
          pltpu.SemaphoreType.DMA,
      ],
  )(flat, rank)


def _grouped_body(meta_ref, xs_ref, gw_ref, uw_ref, dw_ref, ys_ref):
  g = pl.program_id(0)
  t = meta_ref[0, g]
  lo = meta_ref[2, g]
  hi = meta_ref[3, g]
  fv = meta_ref[4, g]

  @pl.when(fv == 1)
  def _():
    ys_ref[...] = jnp.zeros_like(ys_ref)

  @pl.when(hi > lo)
  def _():
    rows = t * M + lax.broadcasted_iota(jnp.int32, (M, 1), 0)
    mask = (rows >= lo) & (rows < hi)
    x = xs_ref[...].astype(jnp.bfloat16)
    gw = gw_ref[0].astype(jnp.bfloat16)
    uw = uw_ref[0].astype(jnp.bfloat16)
    hpre = _dot_nt(x, gw)
    h = (jax.nn.silu(hpre) * _dot_nt(x, uw)).astype(jnp.bfloat16)
    contrib = _dot_nt(h, dw_ref[0].astype(jnp.bfloat16))
    ys_ref[...] += jnp.where(mask, contrib, 0.0)


def _grouped(meta, xs, routed_gate, routed_up, routed_down):
  grid_spec = pltpu.PrefetchScalarGridSpec(
      num_scalar_prefetch=1,
      grid=(G,),
      in_specs=[
          pl.BlockSpec((M, H), lambda g, meta: (meta[0, g], 0)),
          pl.BlockSpec((1, I, H), lambda g, meta: (meta[1, g], 0, 0)),
          pl.BlockSpec((1, I, H), lambda g, meta: (meta[1, g], 0, 0)),
          pl.BlockSpec((1, H, I), lambda g, meta: (meta[1, g], 0, 0)),
      ],
      out_specs=pl.BlockSpec((M, H), lambda g, meta: (meta[0, g], 0)),
  )
  return pl.pallas_call(
      _grouped_body,
      grid_spec=grid_spec,
      out_shape=jax.ShapeDtypeStruct((A, H), jnp.float32),
      compiler_params=pltpu.CompilerParams(
          dimension_semantics=("arbitrary",),
          vmem_limit_bytes=128 * 1024 * 1024,
      ),
  )(meta, xs, routed_gate, routed_up, routed_down)


def _combine_body(ys_hbm, inv1_hbm, inv2_hbm, y1_hbm, y2_hbm,
                  i1_v, i2_v, r1_v, r2_v, sem1, sem2):
  info = plsc.get_sparse_core_info()
  nc = info.num_cores
  wid = lax.axis_index("s") * nc + lax.axis_index("c")
  rpw = N // (nc * info.num_subcores)
  base = wid * rpw
  pltpu.sync_copy(inv1_hbm.at[pl.ds(base, rpw)], i1_v)
  pltpu.sync_copy(inv2_hbm.at[pl.ds(base, rpw)], i2_v)
  cp1 = pltpu.async_copy(ys_hbm.at[i1_v], r1_v, sem1)
  cp2 = pltpu.async_copy(ys_hbm.at[i2_v], r2_v, sem2)
  cp1.wait()
  cp2.wait()
  pltpu.sync_copy(r1_v, y1_hbm.at[pl.ds(base, rpw)])
  pltpu.sync_copy(r2_v, y2_hbm.at[pl.ds(base, rpw)])


def _combine(ys, inv1, inv2):
  info = plsc.get_sparse_core_info()
  rpw = N // (info.num_cores * info.num_subcores)
  mesh = plsc.VectorSubcoreMesh(core_axis_name="c", subcore_axis_name="s")
  return pl.kernel(
      _combine_body,
      out_type=(
          jax.ShapeDtypeStruct((N, H), jnp.float32),
          jax.ShapeDtypeStruct((N, H), jnp.float32),
      ),
      mesh=mesh,
      scratch_types=[
          pltpu.VMEM((rpw,), jnp.int32),
          pltpu.VMEM((rpw,), jnp.int32),
          pltpu.VMEM((rpw, H), jnp.float32),
          pltpu.VMEM((rpw, H), jnp.float32),
          pltpu.SemaphoreType.DMA,
          pltpu.SemaphoreType.DMA,
      ],
  )(ys, inv1, inv2)


MS = 256


def _shared_body(x_ref, sg_ref, su_ref, sd_ref, y1_ref, y2_ref,
                 w1_ref, w2_ref, o_ref):
  t = pl.program_id(0)
  x = x_ref[...]
  h = jax.nn.silu(_dot_nt(x, sg_ref[...])) * _dot_nt(x, su_ref[...])
  out = _dot_nt(h, sd_ref[...])
  w1 = w1_ref[pl.ds(t * MS, MS), :]
  w2 = w2_ref[pl.ds(t * MS, MS), :]
  o_ref[...] = out + w1 * y1_ref[...] + w2 * y2_ref[...]


def _shared_combine(flat, sg, su, sd, ys1, ys2, w1, w2):
  nt = N // MS
  return pl.pallas_call(
      _shared_body,
      grid=(nt,),
      in_specs=[
          pl.BlockSpec((MS, H), lambda t: (t, 0)),
          pl.BlockSpec((I, H), lambda t: (0, 0)),
          pl.BlockSpec((I, H), lambda t: (0, 0)),
          pl.BlockSpec((H, I), lambda t: (0, 0)),
          pl.BlockSpec((MS, H), lambda t: (t, 0)),
          pl.BlockSpec((MS, H), lambda t: (t, 0)),
          pl.BlockSpec((N, 1), lambda t: (0, 0)),
          pl.BlockSpec((N, 1), lambda t: (0, 0)),
      ],
      out_specs=pl.BlockSpec((MS, H), lambda t: (t, 0)),
      out_shape=jax.ShapeDtypeStruct((N, H), jnp.float32),
      compiler_params=pltpu.CompilerParams(
          dimension_semantics=("arbitrary",),
          vmem_limit_bytes=128 * 1024 * 1024,
      ),
  )(flat, sg, su, sd, ys1, ys2, w1, w2)


def kernel(x, shared_gate, shared_up, shared_down, routed_gate, routed_up,
           routed_down, router_w, routing_bias):
  flat = x.reshape(N, H)
  meta, rank, w1, w2 = _plan(flat, router_w, routing_bias)
  rank_flat = rank.reshape(A)
  xs = _dispatch(flat, rank_flat)
  ys = _grouped(meta, xs, routed_gate, routed_up, routed_down)
  ys1, ys2 = _combine(ys, rank_flat[:N], rank_flat[N:])
  out = _shared_combine(flat, shared_gate, shared_up, shared_down,
                        ys1, ys2, w1, w2)
  return out.reshape(x.shape)


# speedup vs baseline: 1.0463x; 1.0163x over previous
import jax
import jax.numpy as jnp
from jax import lax
from jax.experimental import pallas as pl
from jax.experimental.pallas import tpu as pltpu
from jax.experimental.pallas import tpu_sc as plsc

H = 768
I = 1536
E = 63
EP = 64
TOPK = 2
N = 2048
A = N * TOPK
M = 256
T = A // M
G = T + E - 1
C = 128
NCHUNK = A // C

_NT = (((1,), (1,)), ((), ()))


def _dot_nt(a, b):
  return lax.dot_general(a, b, _NT, preferred_element_type=jnp.float32)


def _plan_body(x_ref, rw_ref, b_ref, meta_ref, rank_ref, w1_ref, w2_ref,
               e12_ref):
  x = x_ref[...]
  logits = _dot_nt(x, rw_ref[...])
  logits = logits + b_ref[...]
  probs = jax.nn.sigmoid(logits)
  eidx = lax.broadcasted_iota(jnp.int32, probs.shape, 1)
  big = jnp.int32(2 ** 30)
  m1 = jnp.max(probs, axis=1, keepdims=True)
  i1 = jnp.min(jnp.where(probs == m1, eidx, big), axis=1, keepdims=True)
  probs2 = jnp.where(eidx == i1, -jnp.inf, probs)
  m2 = jnp.max(probs2, axis=1, keepdims=True)
  i2 = jnp.min(jnp.where(probs2 == m2, eidx, big), axis=1, keepdims=True)
  denom = m1 + m2
  w1_ref[...] = m1 / denom
  w2_ref[...] = m2 / denom
  e12_ref[0:N, :] = i1
  e12_ref[N:A, :] = i2

  oh1 = (i1 == eidx).astype(jnp.float32)
  oh2 = (i2 == eidx).astype(jnp.float32)
  counts = (jnp.sum(oh1, axis=0, keepdims=True)
            + jnp.sum(oh2, axis=0, keepdims=True))

  er = lax.broadcasted_iota(jnp.int32, (EP, EP), 0)
  ec = lax.broadcasted_iota(jnp.int32, (EP, EP), 1)
  tri_e = (er < ec).astype(jnp.float32)
  starts = _dot_nt(counts, tri_e.T)
  ends = starts + counts

  rr = lax.broadcasted_iota(jnp.int32, (C, C), 0)
  rc = lax.broadcasted_iota(jnp.int32, (C, C), 1)
  tri_c = (rc < rr).astype(jnp.float32)
  ch_eidx = lax.broadcasted_iota(jnp.int32, (C, EP), 1)

  def chunk(c, running):
    e_ch = e12_ref[pl.ds(c * C, C), :]
    oh = (e_ch == ch_eidx).astype(jnp.float32)
    prefix = lax.dot_general(tri_c, oh, (((1,), (0,)), ((), ())),
                             preferred_element_type=jnp.float32)
    base = starts + running
    r = jnp.sum(oh * (base + prefix), axis=1, keepdims=True)
    rank_ref[pl.ds(c * C, C), :] = r.astype(jnp.int32)
    return running + jnp.sum(oh, axis=0, keepdims=True)

  lax.fori_loop(0, NCHUNK, chunk, jnp.zeros((1, EP), jnp.float32))

  startsc = starts.reshape(EP, 1)
  countsc = counts.reshape(EP, 1)
  endsc = ends.reshape(EP, 1)
  ftile = jnp.floor(startsc / M)
  ltile = jnp.floor((endsc - 1.0) / M)
  ntiles = jnp.where(countsc > 0.0, ltile - ftile + 1.0, 0.0)
  woff = _dot_nt(ntiles.reshape(1, EP), tri_e.T).reshape(EP, 1)
  total = jnp.sum(ntiles)

  gl = lax.broadcasted_iota(jnp.int32, (1, 128), 1).astype(jnp.float32)
  le_mask = (woff <= gl).astype(jnp.float32)
  eg = jnp.sum(le_mask, axis=0, keepdims=True) - 1.0
  ee = lax.broadcasted_iota(jnp.int32, (EP, 128), 0).astype(jnp.float32)
  oh_g = (ee == eg).astype(jnp.float32)
  tg = jnp.sum(oh_g * ftile, axis=0, keepdims=True) + gl - \
      jnp.sum(oh_g * woff, axis=0, keepdims=True)
  lo = jnp.sum(oh_g * startsc, axis=0, keepdims=True)
  hi = jnp.sum(oh_g * endsc, axis=0, keepdims=True)
  valid = (gl < total).astype(jnp.float32)
  lastsel = (gl == total - 1.0).astype(jnp.float32)
  last_t = jnp.sum(lastsel * tg, axis=1, keepdims=True)
  last_e = jnp.sum(lastsel * eg, axis=1, keepdims=True)
  tg = valid * tg + (1.0 - valid) * last_t
  eg = valid * eg + (1.0 - valid) * last_e
  lo = valid * lo
  hi = valid * hi
  tg_prev = jnp.concatenate([jnp.full((1, 1), -1.0, jnp.float32),
                             tg[:, :127]], axis=1)
  fv = valid * (tg != tg_prev).astype(jnp.float32)
  meta = jnp.concatenate(
      [tg, eg, lo, hi, fv, jnp.zeros((3, 128), jnp.float32)], axis=0)
  meta_ref[...] = meta.astype(jnp.int32)


def _plan(flat, router_w, routing_bias):
  rw = jnp.zeros((EP, H), jnp.float32).at[:E].set(router_w)
  bias = jnp.full((1, EP), -jnp.inf, jnp.float32).at[0, :E].set(routing_bias)
  return pl.pallas_call(
      _plan_body,
      out_shape=(
          jax.ShapeDtypeStruct((8, 128), jnp.int32),
          jax.ShapeDtypeStruct((A, 1), jnp.int32),
          jax.ShapeDtypeStruct((N, 1), jnp.float32),
          jax.ShapeDtypeStruct((N, 1), jnp.float32),
      ),
      scratch_shapes=[pltpu.VMEM((A, 1), jnp.int32)],
  )(flat, rw, bias)


def _dispatch_body(flat_hbm, rank_hbm, xs_hbm, idx_v, rows_v, sem):
  info = plsc.get_sparse_core_info()
  nc = info.num_cores
  wid = lax.axis_index("s") * nc + lax.axis_index("c")
  rpw = A // (nc * info.num_subcores)
  base = wid * rpw
  tok_base = lax.rem(base, N)
  pltpu.sync_copy(rank_hbm.at[pl.ds(base, rpw)], idx_v)
  pltpu.sync_copy(flat_hbm.at[pl.ds(tok_base, rpw)], rows_v)
  pltpu.async_copy(rows_v, xs_hbm.at[idx_v], sem).wait()


def _dispatch(flat, rank):
  info = plsc.get_sparse_core_info()
  rpw = A // (info.num_cores * info.num_subcores)
  mesh = plsc.VectorSubcoreMesh(core_axis_name="c", subcore_axis_name="s")
  return pl.kernel(
      _dispatch_body,
      out_type=jax.ShapeDtypeStruct((A, H), jnp.float32),
      mesh=mesh,
      scratch_types=[
          pltpu.VMEM((rpw,), jnp.int32),
          pltpu.VMEM((rpw, H), jnp.float32),
          pltpu.SemaphoreType.DMA,
      ],
  )(flat, rank)


def _grouped_body(meta_ref, xs_ref, gw_ref, uw_ref, dw_ref, ys_ref):
  g = pl.program_id(0)
  t = meta_ref[0, g]
  lo = meta_ref[2, g]
  hi = meta_ref[3, g]
  fv = meta_ref[4, g]

  @pl.when(fv == 1)
  def _():
    ys_ref[...] = jnp.zeros_like(ys_ref)

  @pl.when(hi > lo)
  def _():
    rows = t * M + lax.broadcasted_iota(jnp.int32, (M, 1), 0)
    mask = (rows >= lo) & (rows < hi)
    x = xs_ref[...]
    hpre = _dot_nt(x, gw_ref[0])
    h = jax.nn.silu(hpre) * _dot_nt(x, uw_ref[0])
    contrib = _dot_nt(h, dw_ref[0])
    ys_ref[...] += jnp.where(mask, contrib, 0.0)


def _grouped(meta, xs, routed_gate, routed_up, routed_down):
  grid_spec = pltpu.PrefetchScalarGridSpec(
      num_scalar_prefetch=1,
      grid=(G,),
      in_specs=[
          pl.BlockSpec((M, H), lambda g, meta: (meta[0, g], 0)),
          pl.BlockSpec((1, I, H), lambda g, meta: (meta[1, g], 0, 0)),
          pl.BlockSpec((1, I, H), lambda g, meta: (meta[1, g], 0, 0)),
          pl.BlockSpec((1, H, I), lambda g, meta: (meta[1, g], 0, 0)),
      ],
      out_specs=pl.BlockSpec((M, H), lambda g, meta: (meta[0, g], 0)),
  )
  return pl.pallas_call(
      _grouped_body,
      grid_spec=grid_spec,
      out_shape=jax.ShapeDtypeStruct((A, H), jnp.float32),
      compiler_params=pltpu.CompilerParams(
          dimension_semantics=("arbitrary",),
          vmem_limit_bytes=128 * 1024 * 1024,
      ),
  )(meta, xs, routed_gate, routed_up, routed_down)


def _combine_body(ys_hbm, inv1_hbm, inv2_hbm, y1_hbm, y2_hbm,
                  i1_v, i2_v, r1_v, r2_v, sem1, sem2):
  info = plsc.get_sparse_core_info()
  nc = info.num_cores
  wid = lax.axis_index("s") * nc + lax.axis_index("c")
  rpw = N // (nc * info.num_subcores)
  base = wid * rpw
  pltpu.sync_copy(inv1_hbm.at[pl.ds(base, rpw)], i1_v)
  pltpu.sync_copy(inv2_hbm.at[pl.ds(base, rpw)], i2_v)
  cp1 = pltpu.async_copy(ys_hbm.at[i1_v], r1_v, sem1)
  cp2 = pltpu.async_copy(ys_hbm.at[i2_v], r2_v, sem2)
  cp1.wait()
  cp2.wait()
  pltpu.sync_copy(r1_v, y1_hbm.at[pl.ds(base, rpw)])
  pltpu.sync_copy(r2_v, y2_hbm.at[pl.ds(base, rpw)])


def _combine(ys, inv1, inv2):
  info = plsc.get_sparse_core_info()
  rpw = N // (info.num_cores * info.num_subcores)
  mesh = plsc.VectorSubcoreMesh(core_axis_name="c", subcore_axis_name="s")
  return pl.kernel(
      _combine_body,
      out_type=(
          jax.ShapeDtypeStruct((N, H), jnp.float32),
          jax.ShapeDtypeStruct((N, H), jnp.float32),
      ),
      mesh=mesh,
      scratch_types=[
          pltpu.VMEM((rpw,), jnp.int32),
          pltpu.VMEM((rpw,), jnp.int32),
          pltpu.VMEM((rpw, H), jnp.float32),
          pltpu.VMEM((rpw, H), jnp.float32),
          pltpu.SemaphoreType.DMA,
          pltpu.SemaphoreType.DMA,
      ],
  )(ys, inv1, inv2)


MS = 256


def _shared_body(x_ref, sg_ref, su_ref, sd_ref, y1_ref, y2_ref,
                 w1_ref, w2_ref, o_ref):
  t = pl.program_id(0)
  x = x_ref[...]
  h = jax.nn.silu(_dot_nt(x, sg_ref[...])) * _dot_nt(x, su_ref[...])
  out = _dot_nt(h, sd_ref[...])
  w1 = w1_ref[pl.ds(t * MS, MS), :]
  w2 = w2_ref[pl.ds(t * MS, MS), :]
  o_ref[...] = out + w1 * y1_ref[...] + w2 * y2_ref[...]


def _shared_combine(flat, sg, su, sd, ys1, ys2, w1, w2):
  nt = N // MS
  return pl.pallas_call(
      _shared_body,
      grid=(nt,),
      in_specs=[
          pl.BlockSpec((MS, H), lambda t: (t, 0)),
          pl.BlockSpec((I, H), lambda t: (0, 0)),
          pl.BlockSpec((I, H), lambda t: (0, 0)),
          pl.BlockSpec((H, I), lambda t: (0, 0)),
          pl.BlockSpec((MS, H), lambda t: (t, 0)),
          pl.BlockSpec((MS, H), lambda t: (t, 0)),
          pl.BlockSpec((N, 1), lambda t: (0, 0)),
          pl.BlockSpec((N, 1), lambda t: (0, 0)),
      ],
      out_specs=pl.BlockSpec((MS, H), lambda t: (t, 0)),
      out_shape=jax.ShapeDtypeStruct((N, H), jnp.float32),
      compiler_params=pltpu.CompilerParams(
          dimension_semantics=("arbitrary",),
          vmem_limit_bytes=128 * 1024 * 1024,
      ),
  )(flat, sg, su, sd, ys1, ys2, w1, w2)


def kernel(x, shared_gate, shared_up, shared_down, routed_gate, routed_up,
           routed_down, router_w, routing_bias):
  flat = x.reshape(N, H)
  meta, rank, w1, w2 = _plan(flat, router_w, routing_bias)
  rank_flat = rank.reshape(A)
  xs = _dispatch(flat, rank_flat)
  ys = _grouped(meta, xs, routed_gate, routed_up, routed_down)
  ys1, ys2 = _combine(ys, rank_flat[:N], rank_flat[N:])
  out = _shared_combine(flat, shared_gate, shared_up, shared_down,
                        ys1, ys2, w1, w2)
  return out.reshape(x.shape)


# speedup vs baseline: 1.0565x; 1.0098x over previous
import jax
import jax.numpy as jnp
from jax import lax
from jax.experimental import pallas as pl
from jax.experimental.pallas import tpu as pltpu
from jax.experimental.pallas import tpu_sc as plsc

H = 768
I = 1536
E = 63
EP = 64
TOPK = 2
N = 2048
A = N * TOPK
M = 256
T = A // M
G = T + E - 1
C = 256
NCHUNK = A // C

_NT = (((1,), (1,)), ((), ()))


def _dot_nt(a, b):
  return lax.dot_general(a, b, _NT, preferred_element_type=jnp.float32)


def _plan_body(x_ref, rw_ref, b_ref, meta_ref, rank_ref, w1_ref, w2_ref,
               e12_ref):
  x = x_ref[...]
  logits = _dot_nt(x, rw_ref[...])
  logits = logits + b_ref[...]
  probs = jax.nn.sigmoid(logits)
  eidx = lax.broadcasted_iota(jnp.int32, probs.shape, 1)
  big = jnp.int32(2 ** 30)
  m1 = jnp.max(probs, axis=1, keepdims=True)
  i1 = jnp.min(jnp.where(probs == m1, eidx, big), axis=1, keepdims=True)
  probs2 = jnp.where(eidx == i1, -jnp.inf, probs)
  m2 = jnp.max(probs2, axis=1, keepdims=True)
  i2 = jnp.min(jnp.where(probs2 == m2, eidx, big), axis=1, keepdims=True)
  denom = m1 + m2
  w1_ref[...] = m1 / denom
  w2_ref[...] = m2 / denom
  e12_ref[0:N, :] = i1
  e12_ref[N:A, :] = i2

  oh1 = (i1 == eidx).astype(jnp.float32)
  oh2 = (i2 == eidx).astype(jnp.float32)
  counts = (jnp.sum(oh1, axis=0, keepdims=True)
            + jnp.sum(oh2, axis=0, keepdims=True))

  er = lax.broadcasted_iota(jnp.int32, (EP, EP), 0)
  ec = lax.broadcasted_iota(jnp.int32, (EP, EP), 1)
  tri_e = (er < ec).astype(jnp.float32)
  starts = _dot_nt(counts, tri_e.T)
  ends = starts + counts

  rr = lax.broadcasted_iota(jnp.int32, (C, C), 0)
  rc = lax.broadcasted_iota(jnp.int32, (C, C), 1)
  tri_c = (rc < rr).astype(jnp.float32)
  ch_eidx = lax.broadcasted_iota(jnp.int32, (C, EP), 1)

  def chunk(c, running):
    e_ch = e12_ref[pl.ds(c * C, C), :]
    oh = (e_ch == ch_eidx).astype(jnp.float32)
    prefix = lax.dot_general(tri_c, oh, (((1,), (0,)), ((), ())),
                             preferred_element_type=jnp.float32)
    base = starts + running
    r = jnp.sum(oh * (base + prefix), axis=1, keepdims=True)
    rank_ref[pl.ds(c * C, C), :] = r.astype(jnp.int32)
    return running + jnp.sum(oh, axis=0, keepdims=True)

  lax.fori_loop(0, NCHUNK, chunk, jnp.zeros((1, EP), jnp.float32))

  startsc = starts.reshape(EP, 1)
  countsc = counts.reshape(EP, 1)
  endsc = ends.reshape(EP, 1)
  ftile = jnp.floor(startsc / M)
  ltile = jnp.floor((endsc - 1.0) / M)
  ntiles = jnp.where(countsc > 0.0, ltile - ftile + 1.0, 0.0)
  woff = _dot_nt(ntiles.reshape(1, EP), tri_e.T).reshape(EP, 1)
  total = jnp.sum(ntiles)

  gl = lax.broadcasted_iota(jnp.int32, (1, 128), 1).astype(jnp.float32)
  le_mask = (woff <= gl).astype(jnp.float32)
  eg = jnp.sum(le_mask, axis=0, keepdims=True) - 1.0
  ee = lax.broadcasted_iota(jnp.int32, (EP, 128), 0).astype(jnp.float32)
  oh_g = (ee == eg).astype(jnp.float32)
  tg = jnp.sum(oh_g * ftile, axis=0, keepdims=True) + gl - \
      jnp.sum(oh_g * woff, axis=0, keepdims=True)
  lo = jnp.sum(oh_g * startsc, axis=0, keepdims=True)
  hi = jnp.sum(oh_g * endsc, axis=0, keepdims=True)
  valid = (gl < total).astype(jnp.float32)
  lastsel = (gl == total - 1.0).astype(jnp.float32)
  last_t = jnp.sum(lastsel * tg, axis=1, keepdims=True)
  last_e = jnp.sum(lastsel * eg, axis=1, keepdims=True)
  tg = valid * tg + (1.0 - valid) * last_t
  eg = valid * eg + (1.0 - valid) * last_e
  lo = valid * lo
  hi = valid * hi
  tg_prev = jnp.concatenate([jnp.full((1, 1), -1.0, jnp.float32),
                             tg[:, :127]], axis=1)
  fv = valid * (tg != tg_prev).astype(jnp.float32)
  meta = jnp.concatenate(
      [tg, eg, lo, hi, fv, jnp.zeros((3, 128), jnp.float32)], axis=0)
  meta_ref[...] = meta.astype(jnp.int32)


def _plan(flat, router_w, routing_bias):
  rw = jnp.zeros((EP, H), jnp.float32).at[:E].set(router_w)
  bias = jnp.full((1, EP), -jnp.inf, jnp.float32).at[0, :E].set(routing_bias)
  return pl.pallas_call(
      _plan_body,
      out_shape=(
          jax.ShapeDtypeStruct((8, 128), jnp.int32),
          jax.ShapeDtypeStruct((A, 1), jnp.int32),
          jax.ShapeDtypeStruct((N, 1), jnp.float32),
          jax.ShapeDtypeStruct((N, 1), jnp.float32),
      ),
      scratch_shapes=[pltpu.VMEM((A, 1), jnp.int32)],
  )(flat, rw, bias)


def _dispatch_body(flat_hbm, rank_hbm, xs_hbm, idx_v, rows_v, sem):
  info = plsc.get_sparse_core_info()
  nc = info.num_cores
  wid = lax.axis_index("s") * nc + lax.axis_index("c")
  rpw = A // (nc * info.num_subcores)
  base = wid * rpw
  tok_base = lax.rem(base, N)
  pltpu.sync_copy(rank_hbm.at[pl.ds(base, rpw)], idx_v)
  pltpu.sync_copy(flat_hbm.at[pl.ds(tok_base, rpw)], rows_v)
  pltpu.async_copy(rows_v, xs_hbm.at[idx_v], sem).wait()


def _dispatch(flat, rank):
  info = plsc.get_sparse_core_info()
  rpw = A // (info.num_cores * info.num_subcores)
  mesh = plsc.VectorSubcoreMesh(core_axis_name="c", subcore_axis_name="s")
  return pl.kernel(
      _dispatch_body,
      out_type=jax.ShapeDtypeStruct((A, H), jnp.float32),
      mesh=mesh,
      scratch_types=[
          pltpu.VMEM((rpw,), jnp.int32),
          pltpu.VMEM((rpw, H), jnp.float32),
          pltpu.SemaphoreType.DMA,
      ],
  )(flat, rank)


def _grouped_body(meta_ref, xs_ref, gw_ref, uw_ref, dw_ref, ys_ref):
  g = pl.program_id(0)
  t = meta_ref[0, g]
  lo = meta_ref[2, g]
  hi = meta_ref[3, g]
  fv = meta_ref[4, g]

  @pl.when(fv == 1)
  def _():
    ys_ref[...] = jnp.zeros_like(ys_ref)

  @pl.when(hi > lo)
  def _():
    rows = t * M + lax.broadcasted_iota(jnp.int32, (M, 1), 0)
    mask = (rows >= lo) & (rows < hi)
    x = xs_ref[...]
    hpre = _dot_nt(x, gw_ref[0])
    h = jax.nn.silu(hpre) * _dot_nt(x, uw_ref[0])
    contrib = _dot_nt(h, dw_ref[0])
    ys_ref[...] += jnp.where(mask, contrib, 0.0)


def _grouped(meta, xs, routed_gate, routed_up, routed_down):
  grid_spec = pltpu.PrefetchScalarGridSpec(
      num_scalar_prefetch=1,
      grid=(G,),
      in_specs=[
          pl.BlockSpec((M, H), lambda g, meta: (meta[0, g], 0)),
          pl.BlockSpec((1, I, H), lambda g, meta: (meta[1, g], 0, 0)),
          pl.BlockSpec((1, I, H), lambda g, meta: (meta[1, g], 0, 0)),
          pl.BlockSpec((1, H, I), lambda g, meta: (meta[1, g], 0, 0)),
      ],
      out_specs=pl.BlockSpec((M, H), lambda g, meta: (meta[0, g], 0)),
  )
  return pl.pallas_call(
      _grouped_body,
      grid_spec=grid_spec,
      out_shape=jax.ShapeDtypeStruct((A, H), jnp.float32),
      compiler_params=pltpu.CompilerParams(
          dimension_semantics=("arbitrary",),
          vmem_limit_bytes=128 * 1024 * 1024,
      ),
  )(meta, xs, routed_gate, routed_up, routed_down)


def _combine_body(ys_hbm, inv1_hbm, inv2_hbm, y1_hbm, y2_hbm,
                  i1_v, i2_v, r1_v, r2_v, sem1, sem2):
  info = plsc.get_sparse_core_info()
  nc = info.num_cores
  wid = lax.axis_index("s") * nc + lax.axis_index("c")
  rpw = N // (nc * info.num_subcores)
  base = wid * rpw
  pltpu.sync_copy(inv1_hbm.at[pl.ds(base, rpw)], i1_v)
  pltpu.sync_copy(inv2_hbm.at[pl.ds(base, rpw)], i2_v)
  cp1 = pltpu.async_copy(ys_hbm.at[i1_v], r1_v, sem1)
  cp2 = pltpu.async_copy(ys_hbm.at[i2_v], r2_v, sem2)
  cp1.wait()
  cp2.wait()
  pltpu.sync_copy(r1_v, y1_hbm.at[pl.ds(base, rpw)])
  pltpu.sync_copy(r2_v, y2_hbm.at[pl.ds(base, rpw)])


def _combine(ys, inv1, inv2):
  info = plsc.get_sparse_core_info()
  rpw = N // (info.num_cores * info.num_subcores)
  mesh = plsc.VectorSubcoreMesh(core_axis_name="c", subcore_axis_name="s")
  return pl.kernel(
      _combine_body,
      out_type=(
          jax.ShapeDtypeStruct((N, H), jnp.float32),
          jax.ShapeDtypeStruct((N, H), jnp.float32),
      ),
      mesh=mesh,
      scratch_types=[
          pltpu.VMEM((rpw,), jnp.int32),
          pltpu.VMEM((rpw,), jnp.int32),
          pltpu.VMEM((rpw, H), jnp.float32),
          pltpu.VMEM((rpw, H), jnp.float32),
          pltpu.SemaphoreType.DMA,
          pltpu.SemaphoreType.DMA,
      ],
  )(ys, inv1, inv2)


MS = 256


def _shared_body(x_ref, sg_ref, su_ref, sd_ref, y1_ref, y2_ref,
                 w1_ref, w2_ref, o_ref):
  t = pl.program_id(0)
  x = x_ref[...]
  h = jax.nn.silu(_dot_nt(x, sg_ref[...])) * _dot_nt(x, su_ref[...])
  out = _dot_nt(h, sd_ref[...])
  w1 = w1_ref[pl.ds(t * MS, MS), :]
  w2 = w2_ref[pl.ds(t * MS, MS), :]
  o_ref[...] = out + w1 * y1_ref[...] + w2 * y2_ref[...]


def _shared_combine(flat, sg, su, sd, ys1, ys2, w1, w2):
  nt = N // MS
  return pl.pallas_call(
      _shared_body,
      grid=(nt,),
      in_specs=[
          pl.BlockSpec((MS, H), lambda t: (t, 0)),
          pl.BlockSpec((I, H), lambda t: (0, 0)),
          pl.BlockSpec((I, H), lambda t: (0, 0)),
          pl.BlockSpec((H, I), lambda t: (0, 0)),
          pl.BlockSpec((MS, H), lambda t: (t, 0)),
          pl.BlockSpec((MS, H), lambda t: (t, 0)),
          pl.BlockSpec((N, 1), lambda t: (0, 0)),
          pl.BlockSpec((N, 1), lambda t: (0, 0)),
      ],
      out_specs=pl.BlockSpec((MS, H), lambda t: (t, 0)),
      out_shape=jax.ShapeDtypeStruct((N, H), jnp.float32),
      compiler_params=pltpu.CompilerParams(
          dimension_semantics=("arbitrary",),
          vmem_limit_bytes=128 * 1024 * 1024,
      ),
  )(flat, sg, su, sd, ys1, ys2, w1, w2)


def kernel(x, shared_gate, shared_up, shared_down, routed_gate, routed_up,
           routed_down, router_w, routing_bias):
  flat = x.reshape(N, H)
  meta, rank, w1, w2 = _plan(flat, router_w, routing_bias)
  rank_flat = rank.reshape(A)
  xs = _dispatch(flat, rank_flat)
  ys = _grouped(meta, xs, routed_gate, routed_up, routed_down)
  ys1, ys2 = _combine(ys, rank_flat[:N], rank_flat[N:])
  out = _shared_combine(flat, shared_gate, shared_up, shared_down,
                        ys1, ys2, w1, w2)
  return out.reshape(x.shape)


# speedup vs baseline: 1.0602x; 1.0034x over previous
import jax
import jax.numpy as jnp
from jax import lax
from jax.experimental import pallas as pl
from jax.experimental.pallas import tpu as pltpu
from jax.experimental.pallas import tpu_sc as plsc

H = 768
I = 1536
E = 63
EP = 64
TOPK = 2
N = 2048
A = N * TOPK
M = 256
T = A // M
G = T + E - 1
C = 512
NCHUNK = A // C

_NT = (((1,), (1,)), ((), ()))


def _dot_nt(a, b):
  return lax.dot_general(a, b, _NT, preferred_element_type=jnp.float32)


def _plan_body(x_ref, rw_ref, b_ref, meta_ref, rank_ref, w1_ref, w2_ref,
               e12_ref):
  x = x_ref[...]
  logits = _dot_nt(x, rw_ref[...])
  logits = logits + b_ref[...]
  probs = jax.nn.sigmoid(logits)
  eidx = lax.broadcasted_iota(jnp.int32, probs.shape, 1)
  big = jnp.int32(2 ** 30)
  m1 = jnp.max(probs, axis=1, keepdims=True)
  i1 = jnp.min(jnp.where(probs == m1, eidx, big), axis=1, keepdims=True)
  probs2 = jnp.where(eidx == i1, -jnp.inf, probs)
  m2 = jnp.max(probs2, axis=1, keepdims=True)
  i2 = jnp.min(jnp.where(probs2 == m2, eidx, big), axis=1, keepdims=True)
  denom = m1 + m2
  w1_ref[...] = m1 / denom
  w2_ref[...] = m2 / denom
  e12_ref[0:N, :] = i1
  e12_ref[N:A, :] = i2

  oh1 = (i1 == eidx).astype(jnp.float32)
  oh2 = (i2 == eidx).astype(jnp.float32)
  counts = (jnp.sum(oh1, axis=0, keepdims=True)
            + jnp.sum(oh2, axis=0, keepdims=True))

  er = lax.broadcasted_iota(jnp.int32, (EP, EP), 0)
  ec = lax.broadcasted_iota(jnp.int32, (EP, EP), 1)
  tri_e = (er < ec).astype(jnp.float32)
  starts = _dot_nt(counts, tri_e.T)
  ends = starts + counts

  rr = lax.broadcasted_iota(jnp.int32, (C, C), 0)
  rc = lax.broadcasted_iota(jnp.int32, (C, C), 1)
  tri_c = (rc < rr).astype(jnp.float32)
  ch_eidx = lax.broadcasted_iota(jnp.int32, (C, EP), 1)

  def chunk(c, running):
    e_ch = e12_ref[pl.ds(c * C, C), :]
    oh = (e_ch == ch_eidx).astype(jnp.float32)
    prefix = lax.dot_general(tri_c, oh, (((1,), (0,)), ((), ())),
                             preferred_element_type=jnp.float32)
    base = starts + running
    r = jnp.sum(oh * (base + prefix), axis=1, keepdims=True)
    rank_ref[pl.ds(c * C, C), :] = r.astype(jnp.int32)
    return running + jnp.sum(oh, axis=0, keepdims=True)

  lax.fori_loop(0, NCHUNK, chunk, jnp.zeros((1, EP), jnp.float32))

  startsc = starts.reshape(EP, 1)
  countsc = counts.reshape(EP, 1)
  endsc = ends.reshape(EP, 1)
  ftile = jnp.floor(startsc / M)
  ltile = jnp.floor((endsc - 1.0) / M)
  ntiles = jnp.where(countsc > 0.0, ltile - ftile + 1.0, 0.0)
  woff = _dot_nt(ntiles.reshape(1, EP), tri_e.T).reshape(EP, 1)
  total = jnp.sum(ntiles)

  gl = lax.broadcasted_iota(jnp.int32, (1, 128), 1).astype(jnp.float32)
  le_mask = (woff <= gl).astype(jnp.float32)
  eg = jnp.sum(le_mask, axis=0, keepdims=True) - 1.0
  ee = lax.broadcasted_iota(jnp.int32, (EP, 128), 0).astype(jnp.float32)
  oh_g = (ee == eg).astype(jnp.float32)
  tg = jnp.sum(oh_g * ftile, axis=0, keepdims=True) + gl - \
      jnp.sum(oh_g * woff, axis=0, keepdims=True)
  lo = jnp.sum(oh_g * startsc, axis=0, keepdims=True)
  hi = jnp.sum(oh_g * endsc, axis=0, keepdims=True)
  valid = (gl < total).astype(jnp.float32)
  lastsel = (gl == total - 1.0).astype(jnp.float32)
  last_t = jnp.sum(lastsel * tg, axis=1, keepdims=True)
  last_e = jnp.sum(lastsel * eg, axis=1, keepdims=True)
  tg = valid * tg + (1.0 - valid) * last_t
  eg = valid * eg + (1.0 - valid) * last_e
  lo = valid * lo
  hi = valid * hi
  tg_prev = jnp.concatenate([jnp.full((1, 1), -1.0, jnp.float32),
                             tg[:, :127]], axis=1)
  fv = valid * (tg != tg_prev).astype(jnp.float32)
  meta = jnp.concatenate(
      [tg, eg, lo, hi, fv, jnp.zeros((3, 128), jnp.float32)], axis=0)
  meta_ref[...] = meta.astype(jnp.int32)


def _plan(flat, router_w, routing_bias):
  rw = jnp.zeros((EP, H), jnp.float32).at[:E].set(router_w)
  bias = jnp.full((1, EP), -jnp.inf, jnp.float32).at[0, :E].set(routing_bias)
  return pl.pallas_call(
      _plan_body,
      out_shape=(
          jax.ShapeDtypeStruct((8, 128), jnp.int32),
          jax.ShapeDtypeStruct((A, 1), jnp.int32),
          jax.ShapeDtypeStruct((N, 1), jnp.float32),
          jax.ShapeDtypeStruct((N, 1), jnp.float32),
      ),
      scratch_shapes=[pltpu.VMEM((A, 1), jnp.int32)],
  )(flat, rw, bias)


def _dispatch_body(flat_hbm, rank_hbm, xs_hbm, idx_v, rows_v, sem):
  info = plsc.get_sparse_core_info()
  nc = info.num_cores
  wid = lax.axis_index("s") * nc + lax.axis_index("c")
  rpw = A // (nc * info.num_subcores)
  base = wid * rpw
  tok_base = lax.rem(base, N)
  pltpu.sync_copy(rank_hbm.at[pl.ds(base, rpw)], idx_v)
  pltpu.sync_copy(flat_hbm.at[pl.ds(tok_base, rpw)], rows_v)
  pltpu.async_copy(rows_v, xs_hbm.at[idx_v], sem).wait()


def _dispatch(flat, rank):
  info = plsc.get_sparse_core_info()
  rpw = A // (info.num_cores * info.num_subcores)
  mesh = plsc.VectorSubcoreMesh(core_axis_name="c", subcore_axis_name="s")
  return pl.kernel(
      _dispatch_body,
      out_type=jax.ShapeDtypeStruct((A, H), jnp.float32),
      mesh=mesh,
      scratch_types=[
          pltpu.VMEM((rpw,), jnp.int32),
          pltpu.VMEM((rpw, H), jnp.float32),
          pltpu.SemaphoreType.DMA,
      ],
  )(flat, rank)


def _grouped_body(meta_ref, xs_ref, gw_ref, uw_ref, dw_ref, ys_ref):
  g = pl.program_id(0)
  t = meta_ref[0, g]
  lo = meta_ref[2, g]
  hi = meta_ref[3, g]
  fv = meta_ref[4, g]

  @pl.when(fv == 1)
  def _():
    ys_ref[...] = jnp.zeros_like(ys_ref)

  @pl.when(hi > lo)
  def _():
    rows = t * M + lax.broadcasted_iota(jnp.int32, (M, 1), 0)
    mask = (rows >= lo) & (rows < hi)
    x = xs_ref[...]
    hpre = _dot_nt(x, gw_ref[0])
    h = jax.nn.silu(hpre) * _dot_nt(x, uw_ref[0])
    contrib = _dot_nt(h, dw_ref[0])
    ys_ref[...] += jnp.where(mask, contrib, 0.0)


def _grouped(meta, xs, routed_gate, routed_up, routed_down):
  grid_spec = pltpu.PrefetchScalarGridSpec(
      num_scalar_prefetch=1,
      grid=(G,),
      in_specs=[
          pl.BlockSpec((M, H), lambda g, meta: (meta[0, g], 0)),
          pl.BlockSpec((1, I, H), lambda g, meta: (meta[1, g], 0, 0)),
          pl.BlockSpec((1, I, H), lambda g, meta: (meta[1, g], 0, 0)),
          pl.BlockSpec((1, H, I), lambda g, meta: (meta[1, g], 0, 0)),
      ],
      out_specs=pl.BlockSpec((M, H), lambda g, meta: (meta[0, g], 0)),
  )
  return pl.pallas_call(
      _grouped_body,
      grid_spec=grid_spec,
      out_shape=jax.ShapeDtypeStruct((A, H), jnp.float32),
      compiler_params=pltpu.CompilerParams(
          dimension_semantics=("arbitrary",),
          vmem_limit_bytes=128 * 1024 * 1024,
      ),
  )(meta, xs, routed_gate, routed_up, routed_down)


def _combine_body(ys_hbm, inv1_hbm, inv2_hbm, y1_hbm, y2_hbm,
                  i1_v, i2_v, r1_v, r2_v, sem1, sem2):
  info = plsc.get_sparse_core_info()
  nc = info.num_cores
  wid = lax.axis_index("s") * nc + lax.axis_index("c")
  rpw = N // (nc * info.num_subcores)
  base = wid * rpw
  pltpu.sync_copy(inv1_hbm.at[pl.ds(base, rpw)], i1_v)
  pltpu.sync_copy(inv2_hbm.at[pl.ds(base, rpw)], i2_v)
  cp1 = pltpu.async_copy(ys_hbm.at[i1_v], r1_v, sem1)
  cp2 = pltpu.async_copy(ys_hbm.at[i2_v], r2_v, sem2)
  cp1.wait()
  cp2.wait()
  pltpu.sync_copy(r1_v, y1_hbm.at[pl.ds(base, rpw)])
  pltpu.sync_copy(r2_v, y2_hbm.at[pl.ds(base, rpw)])


def _combine(ys, inv1, inv2):
  info = plsc.get_sparse_core_info()
  rpw = N // (info.num_cores * info.num_subcores)
  mesh = plsc.VectorSubcoreMesh(core_axis_name="c", subcore_axis_name="s")
  return pl.kernel(
      _combine_body,
      out_type=(
          jax.ShapeDtypeStruct((N, H), jnp.float32),
          jax.ShapeDtypeStruct((N, H), jnp.float32),
      ),
      mesh=mesh,
      scratch_types=[
          pltpu.VMEM((rpw,), jnp.int32),
          pltpu.VMEM((rpw,), jnp.int32),
          pltpu.VMEM((rpw, H), jnp.float32),
          pltpu.VMEM((rpw, H), jnp.float32),
          pltpu.SemaphoreType.DMA,
          pltpu.SemaphoreType.DMA,
      ],
  )(ys, inv1, inv2)


MS = 256


def _shared_body(x_ref, sg_ref, su_ref, sd_ref, y1_ref, y2_ref,
                 w1_ref, w2_ref, o_ref):
  t = pl.program_id(0)
  x = x_ref[...]
  h = jax.nn.silu(_dot_nt(x, sg_ref[...])) * _dot_nt(x, su_ref[...])
  out = _dot_nt(h, sd_ref[...])
  w1 = w1_ref[pl.ds(t * MS, MS), :]
  w2 = w2_ref[pl.ds(t * MS, MS), :]
  o_ref[...] = out + w1 * y1_ref[...] + w2 * y2_ref[...]


def _shared_combine(flat, sg, su, sd, ys1, ys2, w1, w2):
  nt = N // MS
  return pl.pallas_call(
      _shared_body,
      grid=(nt,),
      in_specs=[
          pl.BlockSpec((MS, H), lambda t: (t, 0)),
          pl.BlockSpec((I, H), lambda t: (0, 0)),
          pl.BlockSpec((I, H), lambda t: (0, 0)),
          pl.BlockSpec((H, I), lambda t: (0, 0)),
          pl.BlockSpec((MS, H), lambda t: (t, 0)),
          pl.BlockSpec((MS, H), lambda t: (t, 0)),
          pl.BlockSpec((N, 1), lambda t: (0, 0)),
          pl.BlockSpec((N, 1), lambda t: (0, 0)),
      ],
      out_specs=pl.BlockSpec((MS, H), lambda t: (t, 0)),
      out_shape=jax.ShapeDtypeStruct((N, H), jnp.float32),
      compiler_params=pltpu.CompilerParams(
          dimension_semantics=("arbitrary",),
          vmem_limit_bytes=128 * 1024 * 1024,
      ),
  )(flat, sg, su, sd, ys1, ys2, w1, w2)


def kernel(x, shared_gate, shared_up, shared_down, routed_gate, routed_up,
           routed_down, router_w, routing_bias):
  flat = x.reshape(N, H)
  meta, rank, w1, w2 = _plan(flat, router_w, routing_bias)
  rank_flat = rank.reshape(A)
  xs = _dispatch(flat, rank_flat)
  ys = _grouped(meta, xs, routed_gate, routed_up, routed_down)
  ys1, ys2 = _combine(ys, rank_flat[:N], rank_flat[N:])
  out = _shared_combine(flat, shared_gate, shared_up, shared_down,
                        ys1, ys2, w1, w2)
  return out.reshape(x.shape)
